# baseline (speedup 1.0000x reference)
import jax
import jax.numpy as jnp
from jax.experimental import pallas as pl
from jax.experimental.pallas import tpu as pltpu

_B, _CV, _CE = 2, 5, 3
_D, _H, _W = 64, 64, 64
_N = _D * _H * _W
_DC = 4
_G = _D // _DC

_NQ = 16


def _red(x):
    return jnp.sum(x, axis=0)


def _loss_kernel(vox_ref, ends_ref, tv_ref, te_ref, out_ref, acc_ref):
    i = pl.program_id(0)

    @pl.when(i == 0)
    def _init():
        acc_ref[...] = jnp.zeros_like(acc_ref)

    for b in range(_B):
        tv = tv_ref[b]
        te = te_ref[b]
        maskf = (tv > 0).astype(jnp.float32)

        xs = [vox_ref[b * _CV + c] for c in range(_CV)]
        m = xs[0]
        for c in range(1, _CV):
            m = jnp.maximum(m, xs[c])
        se = jnp.exp(xs[0] - m)
        for c in range(1, _CV):
            se = se + jnp.exp(xs[c] - m)
        lse = m + jnp.log(se)

        for c in range(_CV):
            eq = tv == c
            acc_ref[c] = acc_ref[c] + _red(jnp.where(eq, 1.0, 0.0))
            acc_ref[_CV + c] = acc_ref[_CV + c] + _red(
                jnp.where(eq, lse - xs[c], 0.0))

        ys = [ends_ref[b * _CE + c] for c in range(_CE)]
        me = jnp.maximum(jnp.maximum(ys[0], ys[1]), ys[2])
        see = jnp.exp(ys[0] - me) + jnp.exp(ys[1] - me) + jnp.exp(ys[2] - me)
        lsee = me + jnp.log(see)

        for c in range(_CE):
            eqm = (te == c) & (tv > 0)
            acc_ref[10 + c] = acc_ref[10 + c] + _red(
                jnp.where(eqm, 1.0, 0.0))
            acc_ref[13 + c] = acc_ref[13 + c] + _red(
                jnp.where(eqm, lsee - ys[c], 0.0))

    @pl.when(i == _G - 1)
    def _finish():
        s = [jnp.sum(acc_ref[q]) for q in range(_NQ)]
        total = float(_B * _N)
        wv = [1.0 - s[c] / total + 1e-5 for c in range(_CV)]
        num_v = wv[0] * s[5]
        den_v = wv[0] * s[0]
        for c in range(1, _CV):
            num_v = num_v + wv[c] * s[5 + c]
            den_v = den_v + wv[c] * s[c]
        nsel = s[10] + s[11] + s[12]
        we = [1.0 - s[10 + c] / nsel + 1e-5 for c in range(_CE)]
        num_e = we[0] * s[13]
        den_e = we[0] * s[10]
        for c in range(1, _CE):
            num_e = num_e + we[c] * s[13 + c]
            den_e = den_e + we[c] * s[10 + c]
        loss = num_v / den_v + num_e / den_e
        out_ref[...] = jnp.full((1, 1), loss, jnp.float32)


def kernel(input_vox, input_ends, target_vox, target_ends):
    vox = input_vox.reshape(_B * _CV, _D, _H, _W)
    ends = input_ends.reshape(_B * _CE, _D, _H, _W)

    out = pl.pallas_call(
        _loss_kernel,
        grid=(_G,),
        in_specs=[
            pl.BlockSpec((_B * _CV, _DC, _H, _W), lambda i: (0, i, 0, 0)),
            pl.BlockSpec((_B * _CE, _DC, _H, _W), lambda i: (0, i, 0, 0)),
            pl.BlockSpec((_B, _DC, _H, _W), lambda i: (0, i, 0, 0)),
            pl.BlockSpec((_B, _DC, _H, _W), lambda i: (0, i, 0, 0)),
        ],
        out_specs=pl.BlockSpec((1, 1), lambda i: (0, 0)),
        out_shape=jax.ShapeDtypeStruct((1, 1), jnp.float32),
        scratch_shapes=[pltpu.VMEM((_NQ, _H, _W), jnp.float32)],
        compiler_params=pltpu.CompilerParams(
            dimension_semantics=("arbitrary",)),
    )(vox, ends, target_vox, target_ends)
    return out[0, 0]


# speedup vs baseline: 1.1817x; 1.1817x over previous
import jax
import jax.numpy as jnp
from jax.experimental import pallas as pl
from jax.experimental.pallas import tpu as pltpu

_B, _CV, _CE = 2, 5, 3
_D, _H, _W = 64, 64, 64
_N = _D * _H * _W
_DC = 8
_G = _D // _DC

_NQ = 16
_SKIP = (4, 12)


def _fold(x):
    return jnp.sum(x.reshape(8, 8, _W), axis=0)


def _loss_kernel(vox_ref, ends_ref, tv_ref, te_ref, out_ref, acc_ref):
    i = pl.program_id(0)

    @pl.when(i == 0)
    def _init():
        acc_ref[...] = jnp.zeros_like(acc_ref)

    accs = [None if q in _SKIP else jnp.zeros((8, _W), jnp.float32)
            for q in range(_NQ)]

    for b in range(_B):
        for d in range(_DC):
            tv = tv_ref[b, d]
            te = te_ref[b, d]
            msk = tv > 0

            xs = [vox_ref[b * _CV + c, d] for c in range(_CV)]
            m = xs[0]
            for c in range(1, _CV):
                m = jnp.maximum(m, xs[c])
            se = jnp.exp(xs[0] - m)
            for c in range(1, _CV):
                se = se + jnp.exp(xs[c] - m)
            lse = m + jnp.log(se)

            for c in range(_CV):
                eq = tv == c
                if c != 4:
                    accs[c] = accs[c] + _fold(jnp.where(eq, 1.0, 0.0))
                accs[_CV + c] = accs[_CV + c] + _fold(
                    jnp.where(eq, lse - xs[c], 0.0))

            ys = [ends_ref[b * _CE + c, d] for c in range(_CE)]
            me = jnp.maximum(jnp.maximum(ys[0], ys[1]), ys[2])
            see = (jnp.exp(ys[0] - me) + jnp.exp(ys[1] - me)
                   + jnp.exp(ys[2] - me))
            lsee = me + jnp.log(see)

            for c in range(_CE):
                eqm = (te == c) & msk
                if c != 2:
                    accs[10 + c] = accs[10 + c] + _fold(
                        jnp.where(eqm, 1.0, 0.0))
                accs[13 + c] = accs[13 + c] + _fold(
                    jnp.where(eqm, lsee - ys[c], 0.0))

    for q in range(_NQ):
        if q not in _SKIP:
            acc_ref[q] = acc_ref[q] + accs[q]

    @pl.when(i == _G - 1)
    def _finish():
        s = [0.0 if q in _SKIP else jnp.sum(acc_ref[q]) for q in range(_NQ)]
        total = float(_B * _N)
        s[4] = total - (s[0] + s[1] + s[2] + s[3])
        nsel = total - s[0]
        s[12] = nsel - (s[10] + s[11])
        wv = [1.0 - s[c] / total + 1e-5 for c in range(_CV)]
        num_v = wv[0] * s[5]
        den_v = wv[0] * s[0]
        for c in range(1, _CV):
            num_v = num_v + wv[c] * s[5 + c]
            den_v = den_v + wv[c] * s[c]
        we = [1.0 - s[10 + c] / nsel + 1e-5 for c in range(_CE)]
        num_e = we[0] * s[13]
        den_e = we[0] * s[10]
        for c in range(1, _CE):
            num_e = num_e + we[c] * s[13 + c]
            den_e = den_e + we[c] * s[10 + c]
        loss = num_v / den_v + num_e / den_e
        out_ref[...] = jnp.full((1, 1), loss, jnp.float32)


def kernel(input_vox, input_ends, target_vox, target_ends):
    vox = input_vox.reshape(_B * _CV, _D, _H, _W)
    ends = input_ends.reshape(_B * _CE, _D, _H, _W)

    out = pl.pallas_call(
        _loss_kernel,
        grid=(_G,),
        in_specs=[
            pl.BlockSpec((_B * _CV, _DC, _H, _W), lambda i: (0, i, 0, 0)),
            pl.BlockSpec((_B * _CE, _DC, _H, _W), lambda i: (0, i, 0, 0)),
            pl.BlockSpec((_B, _DC, _H, _W), lambda i: (0, i, 0, 0)),
            pl.BlockSpec((_B, _DC, _H, _W), lambda i: (0, i, 0, 0)),
        ],
        out_specs=pl.BlockSpec((1, 1), lambda i: (0, 0)),
        out_shape=jax.ShapeDtypeStruct((1, 1), jnp.float32),
        scratch_shapes=[pltpu.VMEM((_NQ, 8, _W), jnp.float32)],
        compiler_params=pltpu.CompilerParams(
            dimension_semantics=("arbitrary",)),
    )(vox, ends, target_vox, target_ends)
    return out[0, 0]


# speedup vs baseline: 1.1847x; 1.0025x over previous
import jax
import jax.numpy as jnp
from jax.experimental import pallas as pl
from jax.experimental.pallas import tpu as pltpu

_B, _CV, _CE = 2, 5, 3
_D, _H, _W = 64, 64, 64
_N = _D * _H * _W
_DC = 16
_G = _D // _DC

_NQ = 16
_SKIP = (4, 12)


def _fold(x):
    return jnp.sum(x.reshape(8, 8, _W), axis=0)


def _loss_kernel(vox_ref, ends_ref, tv_ref, te_ref, out_ref, acc_ref):
    i = pl.program_id(0)

    @pl.when(i == 0)
    def _init():
        acc_ref[...] = jnp.zeros_like(acc_ref)

    accs = [None if q in _SKIP else jnp.zeros((8, _W), jnp.float32)
            for q in range(_NQ)]

    for b in range(_B):
        for d in range(_DC):
            tv = tv_ref[b, d]
            te = te_ref[b, d]
            msk = tv > 0

            xs = [vox_ref[b * _CV + c, d] for c in range(_CV)]
            m = xs[0]
            for c in range(1, _CV):
                m = jnp.maximum(m, xs[c])
            se = jnp.exp(xs[0] - m)
            for c in range(1, _CV):
                se = se + jnp.exp(xs[c] - m)
            lse = m + jnp.log(se)

            for c in range(_CV):
                eq = tv == c
                if c != 4:
                    accs[c] = accs[c] + _fold(jnp.where(eq, 1.0, 0.0))
                accs[_CV + c] = accs[_CV + c] + _fold(
                    jnp.where(eq, lse - xs[c], 0.0))

            ys = [ends_ref[b * _CE + c, d] for c in range(_CE)]
            me = jnp.maximum(jnp.maximum(ys[0], ys[1]), ys[2])
            see = (jnp.exp(ys[0] - me) + jnp.exp(ys[1] - me)
                   + jnp.exp(ys[2] - me))
            lsee = me + jnp.log(see)

            for c in range(_CE):
                eqm = (te == c) & msk
                if c != 2:
                    accs[10 + c] = accs[10 + c] + _fold(
                        jnp.where(eqm, 1.0, 0.0))
                accs[13 + c] = accs[13 + c] + _fold(
                    jnp.where(eqm, lsee - ys[c], 0.0))

    for q in range(_NQ):
        if q not in _SKIP:
            acc_ref[q] = acc_ref[q] + accs[q]

    @pl.when(i == _G - 1)
    def _finish():
        s = [0.0 if q in _SKIP else jnp.sum(acc_ref[q]) for q in range(_NQ)]
        total = float(_B * _N)
        s[4] = total - (s[0] + s[1] + s[2] + s[3])
        nsel = total - s[0]
        s[12] = nsel - (s[10] + s[11])
        wv = [1.0 - s[c] / total + 1e-5 for c in range(_CV)]
        num_v = wv[0] * s[5]
        den_v = wv[0] * s[0]
        for c in range(1, _CV):
            num_v = num_v + wv[c] * s[5 + c]
            den_v = den_v + wv[c] * s[c]
        we = [1.0 - s[10 + c] / nsel + 1e-5 for c in range(_CE)]
        num_e = we[0] * s[13]
        den_e = we[0] * s[10]
        for c in range(1, _CE):
            num_e = num_e + we[c] * s[13 + c]
            den_e = den_e + we[c] * s[10 + c]
        loss = num_v / den_v + num_e / den_e
        out_ref[...] = jnp.full((1, 1), loss, jnp.float32)


def kernel(input_vox, input_ends, target_vox, target_ends):
    vox = input_vox.reshape(_B * _CV, _D, _H, _W)
    ends = input_ends.reshape(_B * _CE, _D, _H, _W)

    out = pl.pallas_call(
        _loss_kernel,
        grid=(_G,),
        in_specs=[
            pl.BlockSpec((_B * _CV, _DC, _H, _W), lambda i: (0, i, 0, 0)),
            pl.BlockSpec((_B * _CE, _DC, _H, _W), lambda i: (0, i, 0, 0)),
            pl.BlockSpec((_B, _DC, _H, _W), lambda i: (0, i, 0, 0)),
            pl.BlockSpec((_B, _DC, _H, _W), lambda i: (0, i, 0, 0)),
        ],
        out_specs=pl.BlockSpec((1, 1), lambda i: (0, 0)),
        out_shape=jax.ShapeDtypeStruct((1, 1), jnp.float32),
        scratch_shapes=[pltpu.VMEM((_NQ, 8, _W), jnp.float32)],
        compiler_params=pltpu.CompilerParams(
            dimension_semantics=("arbitrary",)),
    )(vox, ends, target_vox, target_ends)
    return out[0, 0]
